# baseline (speedup 1.0000x reference)
import jax
import jax.numpy as jnp
from jax import lax
from jax.experimental import pallas as pl
from jax.experimental.pallas import tpu as pltpu
from jax.experimental.pallas import tpu_sc as plsc

N_NODES = 10000
D_FEAT = 256
L = 16
NW = 32
CHUNK = 64
K_CHUNKS = 80
E_PAD = NW * K_CHUNKS * CHUNK
NB = CHUNK // L
NRING = 5


def _norm_body(z1_ref, n_ref):
    z = z1_ref[...]
    n_ref[...] = jnp.sum(z * z, axis=1)


def _edge_body(z1_hbm, z2_hbm, n_hbm, ts_hbm, e0_hbm, e1_hbm, gd_hbm,
               out_hbm, z2_v, n_v, ts_v, e0_v, e1_v, gd_v, out_v, s_v,
               col_v, semA, semB):
    wid = lax.axis_index("s") * 2 + lax.axis_index("c")
    base = wid * K_CHUNKS * CHUNK

    pltpu.sync_copy(z2_hbm, z2_v)
    pltpu.sync_copy(n_hbm, n_v)
    pltpu.sync_copy(ts_hbm, ts_v)
    pltpu.sync_copy(e0_hbm.at[pl.ds(base, K_CHUNKS * CHUNK)], e0_v)
    pltpu.sync_copy(e1_hbm.at[pl.ds(base, K_CHUNKS * CHUNK)], e1_v)
    pltpu.sync_copy(gd_hbm.at[pl.ds(base, K_CHUNKS * CHUNK)], gd_v)
    ts = ts_v[...]
    lane = lax.iota(jnp.int32, L)

    def issue_e0(c):
        pltpu.async_copy(z1_hbm.at[e0_v.at[pl.ds(c * CHUNK, CHUNK)]],
                         s_v.at[lax.rem(c, NRING)],
                         semA.at[lax.rem(c, NRING)])

    def wait_e0_issue_add(c):
        slot = lax.rem(c, NRING)
        pltpu.make_async_copy(z1_hbm.at[e0_v.at[pl.ds(c * CHUNK, CHUNK)]],
                              s_v.at[slot], semA.at[slot]).wait()
        pltpu.async_copy(z1_hbm.at[e1_v.at[pl.ds(c * CHUNK, CHUNK)]],
                         s_v.at[slot], semB.at[slot], add=True)

    for c in range(4):
        issue_e0(c)
    for c in range(2):
        wait_e0_issue_add(c)

    def chunk_body(c, carry):
        cur = lax.rem(c, NRING)

        @pl.when(c + 4 < K_CHUNKS)
        def _():
            issue_e0(c + 4)

        @pl.when(c + 2 < K_CHUNKS)
        def _():
            wait_e0_issue_add(c + 2)

        pltpu.make_async_copy(z1_hbm.at[e1_v.at[pl.ds(c * CHUNK, CHUNK)]],
                              s_v.at[cur], semB.at[cur]).wait()

        @plsc.parallel_loop(0, CHUNK, unroll=2)
        def edge_loop(e):
            accs = []
            for j in range(4):
                x = s_v[cur, e, pl.ds(j * L, L)]
                accs.append(x * x)
            for j in range(4, D_FEAT // L):
                x = s_v[cur, e, pl.ds(j * L, L)]
                accs[j % 4] = accs[j % 4] + x * x
            acc = (accs[0] + accs[1]) + (accs[2] + accs[3])
            plsc.store_scatter(col_v, [lane * CHUNK + e], acc)

        for b in range(NB):
            parts = [col_v[pl.ds(d2 * CHUNK + b * L, L)] for d2 in range(L)]
            while len(parts) > 1:
                parts = [parts[k] + parts[k + 1]
                         for k in range(0, len(parts), 2)]
            e0b = e0_v[pl.ds(c * CHUNK + b * L, L)]
            e1b = e1_v[pl.ds(c * CHUNK + b * L, L)]
            dot = 0.5 * (parts[0] - plsc.load_gather(n_v, [e0b])
                         - plsc.load_gather(n_v, [e1b]))
            vn = (plsc.load_gather(z2_v, [e0b]) +
                  plsc.load_gather(z2_v, [e1b]))
            gdb = gd_v[pl.ds(c * CHUNK + b * L, L)]
            flag = (dot + gdb) * ts >= 0.0
            sig_f = 1.0 / (1.0 + jnp.exp(-dot))
            sig_n = 1.0 / (1.0 + jnp.exp(-vn))
            out_v[pl.ds(c * CHUNK + b * L, L)] = jnp.where(flag, sig_f, sig_n)
        return 0

    lax.fori_loop(0, K_CHUNKS, chunk_body, 0, unroll=False)
    pltpu.sync_copy(out_v, out_hbm.at[pl.ds(base, K_CHUNKS * CHUNK)])


@jax.jit
def _decode(z1, z2f, n, ts16, e0, e1, gd):
    mesh = plsc.VectorSubcoreMesh(core_axis_name="c", subcore_axis_name="s")
    grid_kernel = pl.kernel(
        _edge_body,
        out_type=jax.ShapeDtypeStruct((E_PAD,), jnp.float32),
        mesh=mesh,
        scratch_types=[
            pltpu.VMEM((N_NODES,), jnp.float32),
            pltpu.VMEM((N_NODES,), jnp.float32),
            pltpu.VMEM((L,), jnp.float32),
            pltpu.VMEM((K_CHUNKS * CHUNK,), jnp.int32),
            pltpu.VMEM((K_CHUNKS * CHUNK,), jnp.int32),
            pltpu.VMEM((K_CHUNKS * CHUNK,), jnp.float32),
            pltpu.VMEM((K_CHUNKS * CHUNK,), jnp.float32),
            pltpu.VMEM((NRING, CHUNK, D_FEAT), jnp.float32),
            pltpu.VMEM((L * CHUNK,), jnp.float32),
            pltpu.SemaphoreType.DMA((NRING,)),
            pltpu.SemaphoreType.DMA((NRING,)),
        ],
        compiler_params=pltpu.CompilerParams(needs_layout_passes=False),
    )
    return grid_kernel(z1, z2f, n, ts16, e0, e1, gd)


def kernel(z1, z2, temp, edge_index):
    n_edges = edge_index.shape[1]
    tau = jnp.asarray(temp, dtype=jnp.float32)
    ts16 = jnp.full((L,), jnp.sign(tau), dtype=jnp.float32)

    u = jax.random.uniform(jax.random.key(42), (n_edges, 2),
                           minval=1e-10, maxval=1.0)
    g = -jnp.log(-jnp.log(u))

    pad = E_PAD - n_edges
    e0 = jnp.pad(edge_index[0], (0, pad))
    e1 = jnp.pad(edge_index[1], (0, pad))
    gd = jnp.pad(g[:, 0] - g[:, 1], (0, pad))
    z2f = z2.reshape(-1)

    nrm = pl.pallas_call(
        _norm_body,
        out_shape=jax.ShapeDtypeStruct((N_NODES,), jnp.float32),
    )(z1)

    out = _decode(z1, z2f, nrm, ts16, e0, e1, gd)
    return out[:n_edges]
